# Initial kernel scaffold; baseline (speedup 1.0000x reference)
import jax
import jax.numpy as jnp
from jax.experimental import pallas as pl


def kernel(x, edge_index, edge_attr, u, batch, W1, b1, W2, b2):
    raise NotImplementedError("write your pallas kernel here")


# speedup vs baseline: 18.5559x; 18.5559x over previous
import jax
import jax.numpy as jnp
from jax import lax
from jax.experimental import pallas as pl
from jax.experimental.pallas import tpu as pltpu
from jax.experimental.pallas import tpu_sc as plsc

N = 100000
E = 1600000
B = 512
DF = 128
DE = 16
DU = 64
HID = 256
DOUT = 64

NC = 2
NS = 16
NW = NC * NS
L = 16

ECHUNK = 512
ENCH = E // ECHUNK
EOUTER = (ENCH + NW - 1) // NW
ESUB = ECHUNK // 128

NCHUNK = 160
NNCH = N // NCHUNK
NOUTER = (NNCH + NW - 1) // NW
NSUB = 2
NHALF = NCHUNK // NSUB


def _mesh():
    return plsc.VectorSubcoreMesh(
        core_axis_name="c", subcore_axis_name="s", num_cores=NC, num_subcores=NS
    )


def _sc_body(row_hbm, attr_hbm, batch_hbm, x_hbm, ze_hbm, zn_hbm,
             esum_hbm, ecnt_hbm, nsum_hbm, ncnt_hbm,
             batch_sh, eacc_sh, nacc_sh,
             rows_v, seg_v, attr_v, cnt_e_v, bat_v, seg2_v, x_v, cnt_n_v):
    c = lax.axis_index("c")
    s = lax.axis_index("s")
    wid = s * NC + c
    iota = lax.iota(jnp.int32, L)
    ones = jnp.ones((L,), jnp.float32)
    zf = jnp.zeros((L,), jnp.float32)

    @pl.when(s == 0)
    def _():
        pltpu.sync_copy(batch_hbm, batch_sh)

    srow = s * (B // NS)
    pltpu.sync_copy(ze_hbm.at[pl.ds(srow, B // NS)],
                    eacc_sh.at[pl.ds(srow, B // NS)])
    pltpu.sync_copy(zn_hbm.at[pl.ds(srow, B // NS)],
                    nacc_sh.at[pl.ds(srow, B // NS)])

    def _zero(i, carry):
        cnt_e_v[i] = zf
        cnt_n_v[i] = zf
        return carry

    lax.fori_loop(0, B, _zero, 0)
    plsc.subcore_barrier()

    def _echunk(i, carry):
        k = wid + i * NW

        @pl.when(k < ENCH)
        def _():
            off = k * ECHUNK
            pltpu.sync_copy(row_hbm.at[pl.ds(off, ECHUNK)], rows_v)
            pltpu.sync_copy(attr_hbm.at[pl.ds(off, ECHUNK)], attr_v)
            for t in range(ESUB):
                pltpu.sync_copy(
                    batch_sh.at[rows_v.at[pl.ds(t * 128, 128)]], seg_v.at[t]
                )
            for t in range(ESUB):
                pltpu.sync_copy(
                    attr_v.at[pl.ds(t * 128, 128)],
                    eacc_sh.at[seg_v.at[t]],
                    add=True,
                )
            for t in range(ESUB):
                for g in range(128 // L):
                    seg16 = seg_v[t, pl.ds(g * L, L)]
                    plsc.addupdate_scatter(cnt_e_v, [seg16, iota], ones)

        return carry

    lax.fori_loop(0, EOUTER, _echunk, 0)

    def _nchunk(i, carry):
        k = wid + i * NW

        @pl.when(k < NNCH)
        def _():
            off = k * NCHUNK
            pltpu.sync_copy(batch_hbm.at[pl.ds(off, NCHUNK)], bat_v)
            pltpu.sync_copy(x_hbm.at[pl.ds(off, NCHUNK)], x_v)
            for h in range(NSUB):
                for t in range(NHALF // L):
                    seg2_v[h, pl.ds(t * L, L)] = bat_v[pl.ds(h * NHALF + t * L, L)]
            for h in range(NSUB):
                pltpu.sync_copy(
                    x_v.at[pl.ds(h * NHALF, NHALF)],
                    nacc_sh.at[seg2_v.at[h]],
                    add=True,
                )
            for g in range(NCHUNK // L):
                seg16 = bat_v[pl.ds(g * L, L)]
                plsc.addupdate_scatter(cnt_n_v, [seg16, iota], ones)

        return carry

    lax.fori_loop(0, NOUTER, _nchunk, 0)

    plsc.subcore_barrier()

    pltpu.sync_copy(eacc_sh.at[pl.ds(srow, B // NS)],
                    esum_hbm.at[c, pl.ds(srow, B // NS)])
    pltpu.sync_copy(nacc_sh.at[pl.ds(srow, B // NS)],
                    nsum_hbm.at[c, pl.ds(srow, B // NS)])
    pltpu.sync_copy(cnt_e_v, ecnt_hbm.at[wid])
    pltpu.sync_copy(cnt_n_v, ncnt_hbm.at[wid])


def _finish_body(u_ref, npart, ncnt, epart, ecnt, w1, b1, w2, b2, o_ref):
    nsum = jnp.sum(npart[...], axis=0)
    ncount = jnp.sum(ncnt[...], axis=(0, 2))[:, None]
    esum = jnp.sum(epart[...], axis=0)
    ecount = jnp.sum(ecnt[...], axis=(0, 2))[:, None]
    ninfo = nsum / jnp.maximum(ncount, 1.0)
    einfo = esum / jnp.maximum(ecount, 1.0)
    cat = jnp.concatenate([u_ref[...], ninfo, einfo], axis=1)
    h = jnp.maximum(
        jnp.dot(cat, w1[...], preferred_element_type=jnp.float32) + b1[...], 0.0
    )
    o_ref[...] = jnp.dot(h, w2[...], preferred_element_type=jnp.float32) + b2[...]


def kernel(x, edge_index, edge_attr, u, batch, W1, b1, W2, b2):
    row = edge_index[0]
    ze = jnp.zeros((B, DE), jnp.float32)
    zn = jnp.zeros((B, DF), jnp.float32)

    sc_fn = pl.kernel(
        _sc_body,
        out_type=(
            jax.ShapeDtypeStruct((NC, B, DE), jnp.float32),
            jax.ShapeDtypeStruct((NW, B, L), jnp.float32),
            jax.ShapeDtypeStruct((NC, B, DF), jnp.float32),
            jax.ShapeDtypeStruct((NW, B, L), jnp.float32),
        ),
        mesh=_mesh(),
        compiler_params=pltpu.CompilerParams(
            needs_layout_passes=False, use_tc_tiling_on_sc=False
        ),
        scratch_types=[
            pltpu.VMEM_SHARED((N,), jnp.int32),
            pltpu.VMEM_SHARED((B, DE), jnp.float32),
            pltpu.VMEM_SHARED((B, DF), jnp.float32),
            pltpu.VMEM((ECHUNK,), jnp.int32),
            pltpu.VMEM((ESUB, 128), jnp.int32),
            pltpu.VMEM((ECHUNK, DE), jnp.float32),
            pltpu.VMEM((B, L), jnp.float32),
            pltpu.VMEM((NCHUNK,), jnp.int32),
            pltpu.VMEM((NSUB, NHALF), jnp.int32),
            pltpu.VMEM((NCHUNK, DF), jnp.float32),
            pltpu.VMEM((B, L), jnp.float32),
        ],
    )
    esum, ecnt, nsum, ncnt = sc_fn(row, edge_attr, batch, x, ze, zn)

    out = pl.pallas_call(
        _finish_body,
        out_shape=jax.ShapeDtypeStruct((B, DOUT), jnp.float32),
    )(u, nsum, ncnt, esum, ecnt, W1, b1.reshape(1, HID), W2, b2.reshape(1, DOUT))
    return out


# speedup vs baseline: 21.0676x; 1.1354x over previous
import jax
import jax.numpy as jnp
from jax import lax
from jax.experimental import pallas as pl
from jax.experimental.pallas import tpu as pltpu
from jax.experimental.pallas import tpu_sc as plsc

N = 100000
E = 1600000
B = 512
DF = 128
DE = 16
DU = 64
HID = 256
DOUT = 64

NC = 2
NS = 16
NW = NC * NS
L = 16

EBLK = 3200
ENCH = E // EBLK
EOUTER = (ENCH + NW - 1) // NW
ESUB = EBLK // 128
EGRP = EBLK // L

NCHUNK = 160
NNCH = N // NCHUNK
NOUTER = (NNCH + NW - 1) // NW
NSUB = 2
NHALF = NCHUNK // NSUB

FROWS = EBLK * DE // 128


def _mesh():
    return plsc.VectorSubcoreMesh(
        core_axis_name="c", subcore_axis_name="s", num_cores=NC, num_subcores=NS
    )


def _sc_params():
    return pltpu.CompilerParams(
        needs_layout_passes=False, use_tc_tiling_on_sc=False
    )


def _fmt_body(a_ref, o_ref):
    b = a_ref[...].T
    o_ref[...] = jnp.concatenate(
        [b[FROWS * k:FROWS * (k + 1), :] for k in range(8)], axis=1
    )


def _node_body(x_hbm, batch_hbm, zn_hbm, nsum_hbm, ncnt_hbm,
               nacc_sh, bat_v, seg2_v, x_v, cnt_n_v):
    c = lax.axis_index("c")
    s = lax.axis_index("s")
    wid = s * NC + c
    iota = lax.iota(jnp.int32, L)
    ones = jnp.ones((L,), jnp.float32)
    zf = jnp.zeros((L,), jnp.float32)

    srow = s * (B // NS)
    pltpu.sync_copy(zn_hbm.at[pl.ds(srow, B // NS)],
                    nacc_sh.at[pl.ds(srow, B // NS)])

    def _zero(i, carry):
        cnt_n_v[i] = zf
        return carry

    lax.fori_loop(0, B, _zero, 0)
    plsc.subcore_barrier()

    def _nchunk(i, carry):
        k = wid + i * NW

        @pl.when(k < NNCH)
        def _():
            off = k * NCHUNK
            pltpu.sync_copy(batch_hbm.at[pl.ds(off, NCHUNK)], bat_v)
            pltpu.sync_copy(x_hbm.at[pl.ds(off, NCHUNK)], x_v)
            for h in range(NSUB):
                for t in range(NHALF // L):
                    seg2_v[h, pl.ds(t * L, L)] = bat_v[pl.ds(h * NHALF + t * L, L)]
            for h in range(NSUB):
                pltpu.sync_copy(
                    x_v.at[pl.ds(h * NHALF, NHALF)],
                    nacc_sh.at[seg2_v.at[h]],
                    add=True,
                )
            for g in range(NCHUNK // L):
                seg16 = bat_v[pl.ds(g * L, L)]
                plsc.addupdate_scatter(cnt_n_v, [seg16, iota], ones)

        return carry

    lax.fori_loop(0, NOUTER, _nchunk, 0)
    plsc.subcore_barrier()

    pltpu.sync_copy(nacc_sh.at[pl.ds(srow, B // NS)],
                    nsum_hbm.at[c, pl.ds(srow, B // NS)])
    pltpu.sync_copy(cnt_n_v, ncnt_hbm.at[wid])


def _edge_body(row_hbm, attr_hbm, batch_hbm, ze_hbm, esum_hbm, ecnt_hbm,
               batch_sh, eacc_sh, rows_v, seg_e_v, seg_p_v, attr_v, cnt_e_v):
    c = lax.axis_index("c")
    s = lax.axis_index("s")
    wid = s * NC + c
    iota = lax.iota(jnp.int32, L)
    ones = jnp.ones((L,), jnp.float32)
    zf = jnp.zeros((L,), jnp.float32)
    pattern = FROWS * (iota % 8) + iota // 8

    @pl.when(s == 0)
    def _():
        pltpu.sync_copy(batch_hbm, batch_sh)

    srow = s * (B // NS)
    pltpu.sync_copy(ze_hbm.at[pl.ds(srow, B // NS)],
                    eacc_sh.at[pl.ds(srow, B // NS)])

    def _zero(i, carry):
        cnt_e_v[i] = zf
        return carry

    lax.fori_loop(0, B, _zero, 0)
    plsc.subcore_barrier()

    def _echunk(i, carry):
        kc = wid + i * NW

        @pl.when(kc < ENCH)
        def _():
            off = kc * EBLK
            pltpu.sync_copy(row_hbm.at[pl.ds(off, EBLK)], rows_v)
            pltpu.sync_copy(attr_hbm.at[pl.ds(off, EBLK)], attr_v)
            for t in range(ESUB):
                pltpu.sync_copy(
                    batch_sh.at[rows_v.at[pl.ds(t * 128, 128)]],
                    seg_e_v.at[pl.ds(t * 128, 128)],
                )

            def _grp(g, c2):
                seg16 = seg_e_v[pl.ds(g * L, L)]
                plsc.addupdate_scatter(cnt_e_v, [seg16, iota], ones)
                segp = plsc.load_gather(seg_e_v, [pattern + 2 * g])
                seg_p_v[g // 8, pl.ds((g % 8) * L, L)] = segp
                return c2

            lax.fori_loop(0, EGRP, _grp, 0)
            for t in range(ESUB):
                pltpu.sync_copy(
                    attr_v.at[pl.ds(t * 128, 128)],
                    eacc_sh.at[seg_p_v.at[t]],
                    add=True,
                )

        return carry

    lax.fori_loop(0, EOUTER, _echunk, 0)
    plsc.subcore_barrier()

    pltpu.sync_copy(eacc_sh.at[pl.ds(srow, B // NS)],
                    esum_hbm.at[c, pl.ds(srow, B // NS)])
    pltpu.sync_copy(cnt_e_v, ecnt_hbm.at[wid])


def _finish_body(u_ref, npart, ncnt, epart, ecnt, w1, b1, w2, b2, o_ref):
    nsum = jnp.sum(npart[...], axis=0)
    ncount = jnp.sum(ncnt[...], axis=(0, 2))[:, None]
    esum = jnp.sum(epart[...], axis=0)
    ecount = jnp.sum(ecnt[...], axis=(0, 2))[:, None]
    ninfo = nsum / jnp.maximum(ncount, 1.0)
    einfo = esum / jnp.maximum(ecount, 1.0)
    cat = jnp.concatenate([u_ref[...], ninfo, einfo], axis=1)
    h = jnp.maximum(
        jnp.dot(cat, w1[...], preferred_element_type=jnp.float32) + b1[...], 0.0
    )
    o_ref[...] = jnp.dot(h, w2[...], preferred_element_type=jnp.float32) + b2[...]


def kernel(x, edge_index, edge_attr, u, batch, W1, b1, W2, b2):
    row = edge_index[0]
    ze = jnp.zeros((B, DE), jnp.float32)
    zn = jnp.zeros((B, DF), jnp.float32)

    ea_lin = pl.pallas_call(
        _fmt_body,
        grid=(ENCH,),
        in_specs=[pl.BlockSpec((DE, EBLK), lambda i: (0, i))],
        out_specs=pl.BlockSpec((FROWS, 128), lambda i: (i, 0)),
        out_shape=jax.ShapeDtypeStruct((E * DE // 128, 128), jnp.float32),
    )(edge_attr.T)
    ea_2d = jnp.reshape(ea_lin, (E, DE))

    node_fn = pl.kernel(
        _node_body,
        out_type=(
            jax.ShapeDtypeStruct((NC, B, DF), jnp.float32),
            jax.ShapeDtypeStruct((NW, B, L), jnp.float32),
        ),
        mesh=_mesh(),
        compiler_params=_sc_params(),
        scratch_types=[
            pltpu.VMEM_SHARED((B, DF), jnp.float32),
            pltpu.VMEM((NCHUNK,), jnp.int32),
            pltpu.VMEM((NSUB, NHALF), jnp.int32),
            pltpu.VMEM((NCHUNK, DF), jnp.float32),
            pltpu.VMEM((B, L), jnp.float32),
        ],
    )
    nsum, ncnt = node_fn(x, batch, zn)

    edge_fn = pl.kernel(
        _edge_body,
        out_type=(
            jax.ShapeDtypeStruct((NC, B, DE), jnp.float32),
            jax.ShapeDtypeStruct((NW, B, L), jnp.float32),
        ),
        mesh=_mesh(),
        compiler_params=_sc_params(),
        scratch_types=[
            pltpu.VMEM_SHARED((N,), jnp.int32),
            pltpu.VMEM_SHARED((B, DE), jnp.float32),
            pltpu.VMEM((EBLK,), jnp.int32),
            pltpu.VMEM((EBLK,), jnp.int32),
            pltpu.VMEM((ESUB, 128), jnp.int32),
            pltpu.VMEM((EBLK, DE), jnp.float32),
            pltpu.VMEM((B, L), jnp.float32),
        ],
    )
    esum, ecnt = edge_fn(row, ea_2d, batch, ze)

    out = pl.pallas_call(
        _finish_body,
        out_shape=jax.ShapeDtypeStruct((B, DOUT), jnp.float32),
    )(u, nsum, ncnt, esum, ecnt, W1, b1.reshape(1, HID), W2, b2.reshape(1, DOUT))
    return out


# speedup vs baseline: 25.2106x; 1.1967x over previous
import jax
import jax.numpy as jnp
from jax import lax
from jax.experimental import pallas as pl
from jax.experimental.pallas import tpu as pltpu
from jax.experimental.pallas import tpu_sc as plsc

N = 100000
E = 1600000
B = 512
DF = 128
DE = 16
DU = 64
HID = 256
DOUT = 64

NC = 2
NS = 16
NW = NC * NS
L = 16

EBLK = 3200
ENCH = E // EBLK
EOUTER = (ENCH + NW - 1) // NW
ESUB = EBLK // 128
EGRP = EBLK // L

NCHUNK = 160
NNCH = N // NCHUNK
NOUTER = (NNCH + NW - 1) // NW
NSUB = 2
NHALF = NCHUNK // NSUB

FROWS = EBLK * DE // 128
FSUB = 4
FBLK = EBLK * FSUB
FGRID = E // FBLK


def _mesh():
    return plsc.VectorSubcoreMesh(
        core_axis_name="c", subcore_axis_name="s", num_cores=NC, num_subcores=NS
    )


def _sc_params():
    return pltpu.CompilerParams(
        needs_layout_passes=False, use_tc_tiling_on_sc=False
    )


def _fmt_body(a_ref, ei_ref, o_ref, row_ref):
    for sb in range(FSUB):
        b = a_ref[:, EBLK * sb:EBLK * (sb + 1)].T
        o_ref[pl.ds(FROWS * sb, FROWS), :] = jnp.concatenate(
            [b[FROWS * k:FROWS * (k + 1), :] for k in range(8)], axis=1
        )
    i = pl.program_id(0)
    row_ref[pl.ds(i * FBLK, FBLK)] = ei_ref[0, :]


def _node_body(x_hbm, batch_hbm, zn_hbm, nsum_hbm, ncnt_hbm,
               nacc_sh, bat_v, seg2_v, x_v, cnt_n_v):
    c = lax.axis_index("c")
    s = lax.axis_index("s")
    wid = s * NC + c
    iota = lax.iota(jnp.int32, L)
    ones = jnp.ones((L,), jnp.float32)
    zf = jnp.zeros((L,), jnp.float32)

    srow = s * (B // NS)
    pltpu.sync_copy(zn_hbm.at[pl.ds(srow, B // NS)],
                    nacc_sh.at[pl.ds(srow, B // NS)])

    def _zero(i, carry):
        cnt_n_v[i] = zf
        return carry

    lax.fori_loop(0, B, _zero, 0)
    plsc.subcore_barrier()

    def _nchunk(i, carry):
        k = wid + i * NW

        @pl.when(k < NNCH)
        def _():
            off = k * NCHUNK
            pltpu.sync_copy(batch_hbm.at[pl.ds(off, NCHUNK)], bat_v)
            pltpu.sync_copy(x_hbm.at[pl.ds(off, NCHUNK)], x_v)
            for h in range(NSUB):
                for t in range(NHALF // L):
                    seg2_v[h, pl.ds(t * L, L)] = bat_v[pl.ds(h * NHALF + t * L, L)]
            for h in range(NSUB):
                pltpu.sync_copy(
                    x_v.at[pl.ds(h * NHALF, NHALF)],
                    nacc_sh.at[seg2_v.at[h]],
                    add=True,
                )
            for g in range(NCHUNK // L):
                seg16 = bat_v[pl.ds(g * L, L)]
                plsc.addupdate_scatter(cnt_n_v, [seg16, iota], ones)

        return carry

    lax.fori_loop(0, NOUTER, _nchunk, 0)
    plsc.subcore_barrier()

    pltpu.sync_copy(nacc_sh.at[pl.ds(srow, B // NS)],
                    nsum_hbm.at[c, pl.ds(srow, B // NS)])
    pltpu.sync_copy(cnt_n_v, ncnt_hbm.at[wid])


def _edge_body(row_hbm, attr_hbm, batch_hbm, ze_hbm, esum_hbm, ecnt_hbm,
               batch_sh, eacc_sh, rows_v, seg_e_v, seg_p_v, attr_v, cnt_e_v):
    c = lax.axis_index("c")
    s = lax.axis_index("s")
    wid = s * NC + c
    iota = lax.iota(jnp.int32, L)
    ones = jnp.ones((L,), jnp.float32)
    zf = jnp.zeros((L,), jnp.float32)
    pattern = FROWS * (iota % 8) + iota // 8

    @pl.when(s == 0)
    def _():
        pltpu.sync_copy(batch_hbm, batch_sh)

    srow = s * (B // NS)
    pltpu.sync_copy(ze_hbm.at[pl.ds(srow, B // NS)],
                    eacc_sh.at[pl.ds(srow, B // NS)])

    def _zero(i, carry):
        cnt_e_v[i] = zf
        return carry

    lax.fori_loop(0, B, _zero, 0)
    plsc.subcore_barrier()

    def _echunk(i, carry):
        kc = wid + i * NW

        @pl.when(kc < ENCH)
        def _():
            off = kc * EBLK
            pltpu.sync_copy(row_hbm.at[pl.ds(off, EBLK)], rows_v)
            pltpu.sync_copy(attr_hbm.at[pl.ds(off, EBLK)], attr_v)
            for t in range(ESUB):
                pltpu.sync_copy(
                    batch_sh.at[rows_v.at[pl.ds(t * 128, 128)]],
                    seg_e_v.at[pl.ds(t * 128, 128)],
                )

            def _grp(g, c2):
                seg16 = seg_e_v[pl.ds(g * L, L)]
                plsc.addupdate_scatter(cnt_e_v, [seg16, iota], ones)
                segp = plsc.load_gather(seg_e_v, [pattern + 2 * g])
                seg_p_v[g // 8, pl.ds((g % 8) * L, L)] = segp
                return c2

            lax.fori_loop(0, EGRP, _grp, 0)
            for t in range(ESUB):
                pltpu.sync_copy(
                    attr_v.at[pl.ds(t * 128, 128)],
                    eacc_sh.at[seg_p_v.at[t]],
                    add=True,
                )

        return carry

    lax.fori_loop(0, EOUTER, _echunk, 0)
    plsc.subcore_barrier()

    pltpu.sync_copy(eacc_sh.at[pl.ds(srow, B // NS)],
                    esum_hbm.at[c, pl.ds(srow, B // NS)])
    pltpu.sync_copy(cnt_e_v, ecnt_hbm.at[wid])


def _finish_body(u_ref, npart, ncnt, epart, ecnt, w1, b1, w2, b2, o_ref):
    nsum = jnp.sum(npart[...], axis=0)
    ncount = jnp.sum(ncnt[...], axis=(0, 2))[:, None]
    esum = jnp.sum(epart[...], axis=0)
    ecount = jnp.sum(ecnt[...], axis=(0, 2))[:, None]
    ninfo = nsum / jnp.maximum(ncount, 1.0)
    einfo = esum / jnp.maximum(ecount, 1.0)
    cat = jnp.concatenate([u_ref[...], ninfo, einfo], axis=1)
    h = jnp.maximum(
        jnp.dot(cat, w1[...], preferred_element_type=jnp.float32) + b1[...], 0.0
    )
    o_ref[...] = jnp.dot(h, w2[...], preferred_element_type=jnp.float32) + b2[...]


def kernel(x, edge_index, edge_attr, u, batch, W1, b1, W2, b2):
    ze = jnp.zeros((B, DE), jnp.float32)
    zn = jnp.zeros((B, DF), jnp.float32)

    ea_lin, row = pl.pallas_call(
        _fmt_body,
        grid=(FGRID,),
        in_specs=[
            pl.BlockSpec((DE, FBLK), lambda i: (0, i)),
            pl.BlockSpec((2, FBLK), lambda i: (0, i)),
        ],
        out_specs=[
            pl.BlockSpec((FSUB * FROWS, 128), lambda i: (i, 0)),
            pl.BlockSpec((E,), lambda i: (0,)),
        ],
        out_shape=[
            jax.ShapeDtypeStruct((E * DE // 128, 128), jnp.float32),
            jax.ShapeDtypeStruct((E,), jnp.int32),
        ],
    )(edge_attr.T, edge_index)
    ea_2d = jnp.reshape(ea_lin, (E, DE))

    node_fn = pl.kernel(
        _node_body,
        out_type=(
            jax.ShapeDtypeStruct((NC, B, DF), jnp.float32),
            jax.ShapeDtypeStruct((NW, B, L), jnp.float32),
        ),
        mesh=_mesh(),
        compiler_params=_sc_params(),
        scratch_types=[
            pltpu.VMEM_SHARED((B, DF), jnp.float32),
            pltpu.VMEM((NCHUNK,), jnp.int32),
            pltpu.VMEM((NSUB, NHALF), jnp.int32),
            pltpu.VMEM((NCHUNK, DF), jnp.float32),
            pltpu.VMEM((B, L), jnp.float32),
        ],
    )
    nsum, ncnt = node_fn(x, batch, zn)

    edge_fn = pl.kernel(
        _edge_body,
        out_type=(
            jax.ShapeDtypeStruct((NC, B, DE), jnp.float32),
            jax.ShapeDtypeStruct((NW, B, L), jnp.float32),
        ),
        mesh=_mesh(),
        compiler_params=_sc_params(),
        scratch_types=[
            pltpu.VMEM_SHARED((N,), jnp.int32),
            pltpu.VMEM_SHARED((B, DE), jnp.float32),
            pltpu.VMEM((EBLK,), jnp.int32),
            pltpu.VMEM((EBLK,), jnp.int32),
            pltpu.VMEM((ESUB, 128), jnp.int32),
            pltpu.VMEM((EBLK, DE), jnp.float32),
            pltpu.VMEM((B, L), jnp.float32),
        ],
    )
    esum, ecnt = edge_fn(row, ea_2d, batch, ze)

    out = pl.pallas_call(
        _finish_body,
        out_shape=jax.ShapeDtypeStruct((B, DOUT), jnp.float32),
    )(u, nsum, ncnt, esum, ecnt, W1, b1.reshape(1, HID), W2, b2.reshape(1, DOUT))
    return out


# speedup vs baseline: 32.6731x; 1.2960x over previous
import jax
import jax.numpy as jnp
from jax import lax
from jax.experimental import pallas as pl
from jax.experimental.pallas import tpu as pltpu
from jax.experimental.pallas import tpu_sc as plsc

N = 100000
E = 1600000
B = 512
DF = 128
DE = 16
DU = 64
HID = 256
DOUT = 64

NC = 2
NS = 16
NW = NC * NS
L = 16

EBLK = 3200
ENCH = E // EBLK
EOUTER = (ENCH + NW - 1) // NW
ESUB = EBLK // 128
EGRP = EBLK // L

NCHUNK = 160
NNCH = N // NCHUNK
NOUTER = (NNCH + NW - 1) // NW
NSUB = 2
NHALF = NCHUNK // NSUB

FROWS = EBLK * DE // 128
FSUB = 4
FBLK = EBLK * FSUB
FGRID = E // FBLK


def _mesh():
    return plsc.VectorSubcoreMesh(
        core_axis_name="c", subcore_axis_name="s", num_cores=NC, num_subcores=NS
    )


def _sc_params():
    return pltpu.CompilerParams(
        needs_layout_passes=False, use_tc_tiling_on_sc=False
    )


def _fmt_body(a_ref, ei_ref, o_ref, row_ref):
    for sb in range(FSUB):
        b = a_ref[:, EBLK * sb:EBLK * (sb + 1)].T
        o_ref[pl.ds(FROWS * sb, FROWS), :] = jnp.concatenate(
            [b[FROWS * k:FROWS * (k + 1), :] for k in range(8)], axis=1
        )
    i = pl.program_id(0)
    row_ref[pl.ds(i * FBLK, FBLK)] = ei_ref[0, :]


def _node_body(x_hbm, batch_hbm, zn_hbm, nsum_hbm, ncnt_hbm,
               nacc_sh, bat_v, seg2_v, x_v, cnt_n_v):
    c = lax.axis_index("c")
    s = lax.axis_index("s")
    wid = s * NC + c
    iota = lax.iota(jnp.int32, L)
    ones = jnp.ones((L,), jnp.float32)
    zf = jnp.zeros((L,), jnp.float32)

    srow = s * (B // NS)
    pltpu.sync_copy(zn_hbm.at[pl.ds(srow, B // NS)],
                    nacc_sh.at[pl.ds(srow, B // NS)])

    def _zero(i, carry):
        cnt_n_v[i] = zf
        return carry

    lax.fori_loop(0, B, _zero, 0)
    plsc.subcore_barrier()

    def _nchunk(i, carry):
        k = wid + i * NW

        @pl.when(k < NNCH)
        def _():
            off = k * NCHUNK
            pltpu.sync_copy(batch_hbm.at[pl.ds(off, NCHUNK)], bat_v)
            pltpu.sync_copy(x_hbm.at[pl.ds(off, NCHUNK)], x_v)
            for h in range(NSUB):
                for t in range(NHALF // L):
                    seg2_v[h, pl.ds(t * L, L)] = bat_v[pl.ds(h * NHALF + t * L, L)]
            for h in range(NSUB):
                pltpu.sync_copy(
                    x_v.at[pl.ds(h * NHALF, NHALF)],
                    nacc_sh.at[seg2_v.at[h]],
                    add=True,
                )
            for g in range(NCHUNK // L):
                seg16 = bat_v[pl.ds(g * L, L)]
                plsc.addupdate_scatter(cnt_n_v, [seg16, iota], ones)

        return carry

    lax.fori_loop(0, NOUTER, _nchunk, 0)
    plsc.subcore_barrier()

    pltpu.sync_copy(nacc_sh.at[pl.ds(srow, B // NS)],
                    nsum_hbm.at[c, pl.ds(srow, B // NS)])
    pltpu.sync_copy(cnt_n_v, ncnt_hbm.at[wid])


def _edge_body(row_hbm, attr_hbm, batch_hbm, ze_hbm, esum_hbm, ecnt_hbm,
               batch_sh, eacc_sh, rows_v, seg_e_v, seg_p_v, attr_v, cnt_e_v,
               sem_g, sem_a, sem_s):
    c = lax.axis_index("c")
    s = lax.axis_index("s")
    wid = s * NC + c
    iota = lax.iota(jnp.int32, L)
    ones = jnp.ones((L,), jnp.float32)
    zf = jnp.zeros((L,), jnp.float32)
    pattern = FROWS * (iota % 8) + iota // 8

    @pl.when(s == 0)
    def _():
        pltpu.sync_copy(batch_hbm, batch_sh)

    srow = s * (B // NS)
    pltpu.sync_copy(ze_hbm.at[pl.ds(srow, B // NS)],
                    eacc_sh.at[pl.ds(srow, B // NS)])

    def _zero(i, carry):
        cnt_e_v[i] = zf
        return carry

    lax.fori_loop(0, B, _zero, 0)
    plsc.subcore_barrier()

    def _scatter_drain():
        for t in range(ESUB):
            pltpu.make_async_copy(
                attr_v.at[pl.ds(t * 128, 128)],
                eacc_sh.at[seg_p_v.at[t]],
                sem_s,
            ).wait()

    def _echunk(i, carry):
        kc = wid + i * NW

        @pl.when(kc < ENCH)
        def _():
            off = kc * EBLK
            pltpu.sync_copy(row_hbm.at[pl.ds(off, EBLK)], rows_v)
            for t in range(ESUB):
                pltpu.async_copy(
                    batch_sh.at[rows_v.at[pl.ds(t * 128, 128)]],
                    seg_e_v.at[pl.ds(t * 128, 128)],
                    sem_g,
                )

        @pl.when((i > 0) & (wid + (i - 1) * NW < ENCH))
        def _():
            _scatter_drain()

        @pl.when(kc < ENCH)
        def _():
            off = kc * EBLK
            da = pltpu.async_copy(attr_hbm.at[pl.ds(off, EBLK)], attr_v, sem_a)
            for t in range(ESUB):
                pltpu.make_async_copy(
                    batch_sh.at[rows_v.at[pl.ds(t * 128, 128)]],
                    seg_e_v.at[pl.ds(t * 128, 128)],
                    sem_g,
                ).wait()

            def _grp(g, c2):
                seg16 = seg_e_v[pl.ds(g * L, L)]
                plsc.addupdate_scatter(cnt_e_v, [seg16, iota], ones)
                segp = plsc.load_gather(seg_e_v, [pattern + 2 * g])
                seg_p_v[g // 8, pl.ds((g % 8) * L, L)] = segp
                return c2

            lax.fori_loop(0, EGRP, _grp, 0)
            da.wait()
            for t in range(ESUB):
                pltpu.async_copy(
                    attr_v.at[pl.ds(t * 128, 128)],
                    eacc_sh.at[seg_p_v.at[t]],
                    sem_s,
                )

        return carry

    lax.fori_loop(0, EOUTER, _echunk, 0)

    @pl.when(wid + (EOUTER - 1) * NW < ENCH)
    def _():
        _scatter_drain()

    plsc.subcore_barrier()

    pltpu.sync_copy(eacc_sh.at[pl.ds(srow, B // NS)],
                    esum_hbm.at[c, pl.ds(srow, B // NS)])
    pltpu.sync_copy(cnt_e_v, ecnt_hbm.at[wid])


def _finish_body(u_ref, npart, ncnt, epart, ecnt, w1, b1, w2, b2, o_ref):
    nsum = jnp.sum(npart[...], axis=0)
    ncount = jnp.sum(ncnt[...], axis=(0, 2))[:, None]
    esum = jnp.sum(epart[...], axis=0)
    ecount = jnp.sum(ecnt[...], axis=(0, 2))[:, None]
    ninfo = nsum / jnp.maximum(ncount, 1.0)
    einfo = esum / jnp.maximum(ecount, 1.0)
    cat = jnp.concatenate([u_ref[...], ninfo, einfo], axis=1)
    h = jnp.maximum(
        jnp.dot(cat, w1[...], preferred_element_type=jnp.float32) + b1[...], 0.0
    )
    o_ref[...] = jnp.dot(h, w2[...], preferred_element_type=jnp.float32) + b2[...]


def kernel(x, edge_index, edge_attr, u, batch, W1, b1, W2, b2):
    ze = jnp.zeros((B, DE), jnp.float32)
    zn = jnp.zeros((B, DF), jnp.float32)

    ea_lin, row = pl.pallas_call(
        _fmt_body,
        grid=(FGRID,),
        in_specs=[
            pl.BlockSpec((DE, FBLK), lambda i: (0, i)),
            pl.BlockSpec((2, FBLK), lambda i: (0, i)),
        ],
        out_specs=[
            pl.BlockSpec((FSUB * FROWS, 128), lambda i: (i, 0)),
            pl.BlockSpec((E,), lambda i: (0,)),
        ],
        out_shape=[
            jax.ShapeDtypeStruct((E * DE // 128, 128), jnp.float32),
            jax.ShapeDtypeStruct((E,), jnp.int32),
        ],
    )(edge_attr.T, edge_index)
    ea_2d = jnp.reshape(ea_lin, (E, DE))

    node_fn = pl.kernel(
        _node_body,
        out_type=(
            jax.ShapeDtypeStruct((NC, B, DF), jnp.float32),
            jax.ShapeDtypeStruct((NW, B, L), jnp.float32),
        ),
        mesh=_mesh(),
        compiler_params=_sc_params(),
        scratch_types=[
            pltpu.VMEM_SHARED((B, DF), jnp.float32),
            pltpu.VMEM((NCHUNK,), jnp.int32),
            pltpu.VMEM((NSUB, NHALF), jnp.int32),
            pltpu.VMEM((NCHUNK, DF), jnp.float32),
            pltpu.VMEM((B, L), jnp.float32),
        ],
    )
    nsum, ncnt = node_fn(x, batch, zn)

    edge_fn = pl.kernel(
        _edge_body,
        out_type=(
            jax.ShapeDtypeStruct((NC, B, DE), jnp.float32),
            jax.ShapeDtypeStruct((NW, B, L), jnp.float32),
        ),
        mesh=_mesh(),
        compiler_params=_sc_params(),
        scratch_types=[
            pltpu.VMEM_SHARED((N,), jnp.int32),
            pltpu.VMEM_SHARED((B, DE), jnp.float32),
            pltpu.VMEM((EBLK,), jnp.int32),
            pltpu.VMEM((EBLK,), jnp.int32),
            pltpu.VMEM((ESUB, 128), jnp.int32),
            pltpu.VMEM((EBLK, DE), jnp.float32),
            pltpu.VMEM((B, L), jnp.float32),
            pltpu.SemaphoreType.DMA,
            pltpu.SemaphoreType.DMA,
            pltpu.SemaphoreType.DMA,
        ],
    )
    ze_dep = ze + ncnt[0, :, :DE] * 0.0
    esum, ecnt = edge_fn(row, ea_2d, batch, ze_dep)

    out = pl.pallas_call(
        _finish_body,
        out_shape=jax.ShapeDtypeStruct((B, DOUT), jnp.float32),
    )(u, nsum, ncnt, esum, ecnt, W1, b1.reshape(1, HID), W2, b2.reshape(1, DOUT))
    return out
